# baseline (speedup 1.0000x reference)
import functools
import math

import jax
import jax.numpy as jnp
import numpy as np
from jax import lax
from jax.experimental import pallas as pl
from jax.experimental.pallas import tpu as pltpu
from jax.experimental.pallas import tpu_sc as plsc

VOCAB_SIZE = 1000000
DIM = 64
SEQ_LEN = 200
BATCH = 1024

NUM_WORKERS = 32
TOTAL_ROWS = BATCH * SEQ_LEN
ROWS_PER_WORKER = TOTAL_ROWS // NUM_WORKERS
CHUNK = 128
CHUNKS_PER_WORKER = ROWS_PER_WORKER // CHUNK


def _position_embedding_np():
    i = np.arange(SEQ_LEN, dtype=np.float64)[:, None]
    j = np.arange(DIM, dtype=np.float64)[None, :]
    even_mask = (np.arange(DIM) % 2 == 0)[None, :]
    temp_even = np.exp(-(j / DIM) * math.log(10000.0))
    temp_odd = np.exp(-((j - 1.0) / DIM) * math.log(10000.0))
    pe = np.where(even_mask, np.sin(i * temp_even), np.cos(i * temp_odd))
    return pe.astype(np.float32)


_PE2 = np.concatenate([_position_embedding_np()] * 2, axis=0)


def _sc_body(idx_hbm, table_hbm, pe_hbm, out_hbm, idx_v, pe_v, rows_v, sem):
    wid = lax.axis_index("s") * 2 + lax.axis_index("c")
    wstart = wid * ROWS_PER_WORKER

    pltpu.sync_copy(idx_hbm.at[wid], idx_v)
    pltpu.sync_copy(pe_hbm, pe_v)

    def chunk_body(c, carry):
        base = wstart + c * CHUNK
        off = lax.rem(c * CHUNK, SEQ_LEN)
        pltpu.async_copy(table_hbm.at[idx_v.at[c]], rows_v, sem).wait()

        def row_body(r, carry2):
            for j in range(DIM // 16):
                sl = pl.ds(j * 16, 16)
                rows_v[r, sl] += pe_v[off + r, sl]
            return carry2

        lax.fori_loop(0, CHUNK, row_body, 0, unroll=4)
        pltpu.sync_copy(rows_v, out_hbm.at[pl.ds(base, CHUNK)])
        return carry

    lax.fori_loop(0, CHUNKS_PER_WORKER, chunk_body, 0)


@jax.jit
def kernel(x, table):
    idx = x.reshape(NUM_WORKERS, CHUNKS_PER_WORKER, CHUNK)
    pe2 = jnp.asarray(_PE2)
    run = pl.kernel(
        _sc_body,
        out_type=jax.ShapeDtypeStruct((TOTAL_ROWS, DIM), jnp.float32),
        mesh=plsc.VectorSubcoreMesh(core_axis_name="c", subcore_axis_name="s"),
        scratch_types=[
            pltpu.VMEM((CHUNKS_PER_WORKER, CHUNK), jnp.int32),
            pltpu.VMEM((2 * SEQ_LEN, DIM), jnp.float32),
            pltpu.VMEM((CHUNK, DIM), jnp.float32),
            pltpu.SemaphoreType.DMA,
        ],
        compiler_params=pltpu.CompilerParams(use_tc_tiling_on_sc=False),
    )
    out = run(idx, table, pe2)
    return out.reshape(BATCH, SEQ_LEN, DIM)


# speedup vs baseline: 1.0367x; 1.0367x over previous
import functools
import math

import jax
import jax.numpy as jnp
import numpy as np
from jax import lax
from jax.experimental import pallas as pl
from jax.experimental.pallas import tpu as pltpu
from jax.experimental.pallas import tpu_sc as plsc

VOCAB_SIZE = 1000000
DIM = 64
SEQ_LEN = 200
BATCH = 1024

NUM_WORKERS = 32
TOTAL_ROWS = BATCH * SEQ_LEN
ROWS_PER_WORKER = TOTAL_ROWS // NUM_WORKERS
CHUNK = 128
CHUNKS_PER_WORKER = ROWS_PER_WORKER // CHUNK


def _position_embedding_np():
    i = np.arange(SEQ_LEN, dtype=np.float64)[:, None]
    j = np.arange(DIM, dtype=np.float64)[None, :]
    even_mask = (np.arange(DIM) % 2 == 0)[None, :]
    temp_even = np.exp(-(j / DIM) * math.log(10000.0))
    temp_odd = np.exp(-((j - 1.0) / DIM) * math.log(10000.0))
    pe = np.where(even_mask, np.sin(i * temp_even), np.cos(i * temp_odd))
    return pe.astype(np.float32)


_PE = _position_embedding_np()

_POS = ((np.arange(CHUNKS_PER_WORKER * CHUNK) % SEQ_LEN)
        .astype(np.int32).reshape(CHUNKS_PER_WORKER, CHUNK))


NBUF = 5
N_CHUNKS = CHUNKS_PER_WORKER


def _sc_body(idx_hbm, pos_hbm, table_hbm, pe_hbm, out_hbm,
             idx_v, pos_v, rows_v, gsem, psem, wsem):
    wid = lax.axis_index("s") * 2 + lax.axis_index("c")
    wstart = wid * ROWS_PER_WORKER

    pltpu.sync_copy(idx_hbm.at[wid], idx_v)
    pltpu.sync_copy(pos_hbm, pos_v)

    def slot(g):
        return lax.rem(g, NBUF)

    def fire_tbl(g):
        pltpu.async_copy(table_hbm.at[idx_v.at[g]], rows_v.at[slot(g)],
                         gsem.at[slot(g)])

    def wait_tbl(g):
        pltpu.make_async_copy(table_hbm.at[idx_v.at[g]], rows_v.at[slot(g)],
                              gsem.at[slot(g)]).wait()

    def fire_pe(g):
        pltpu.async_copy(pe_hbm.at[pos_v.at[g]], rows_v.at[slot(g)],
                         psem.at[slot(g)], add=True)

    def wait_pe(g):
        pltpu.make_async_copy(pe_hbm.at[pos_v.at[g]], rows_v.at[slot(g)],
                              psem.at[slot(g)]).wait()

    def fire_write(g):
        pltpu.async_copy(rows_v.at[slot(g)],
                         out_hbm.at[pl.ds(wstart + g * CHUNK, CHUNK)],
                         wsem.at[slot(g)])

    def wait_write(g):
        pltpu.make_async_copy(rows_v.at[slot(g)],
                              out_hbm.at[pl.ds(wstart + g * CHUNK, CHUNK)],
                              wsem.at[slot(g)]).wait()

    for i in range(NBUF):
        fire_tbl(i)
        if i >= 1:
            wait_tbl(i - 1)
            fire_pe(i - 1)
        if i >= 2:
            wait_pe(i - 2)
            fire_write(i - 2)

    def steady(i, carry):
        wait_write(i - NBUF)
        fire_tbl(i)
        wait_tbl(i - 1)
        fire_pe(i - 1)
        wait_pe(i - 2)
        fire_write(i - 2)
        return carry

    lax.fori_loop(NBUF, N_CHUNKS, steady, 0)

    wait_tbl(N_CHUNKS - 1)
    fire_pe(N_CHUNKS - 1)
    wait_pe(N_CHUNKS - 2)
    fire_write(N_CHUNKS - 2)
    wait_pe(N_CHUNKS - 1)
    fire_write(N_CHUNKS - 1)
    for g in range(N_CHUNKS - NBUF, N_CHUNKS):
        wait_write(g)


@jax.jit
def kernel(x, table):
    idx = x.reshape(NUM_WORKERS, CHUNKS_PER_WORKER, CHUNK)
    pe = jnp.asarray(_PE)
    pos = jnp.asarray(_POS)
    run = pl.kernel(
        _sc_body,
        out_type=jax.ShapeDtypeStruct((TOTAL_ROWS, DIM), jnp.float32),
        mesh=plsc.VectorSubcoreMesh(core_axis_name="c", subcore_axis_name="s"),
        scratch_types=[
            pltpu.VMEM((CHUNKS_PER_WORKER, CHUNK), jnp.int32),
            pltpu.VMEM((CHUNKS_PER_WORKER, CHUNK), jnp.int32),
            pltpu.VMEM((NBUF, CHUNK, DIM), jnp.float32),
            pltpu.SemaphoreType.DMA((NBUF,)),
            pltpu.SemaphoreType.DMA((NBUF,)),
            pltpu.SemaphoreType.DMA((NBUF,)),
        ],
        compiler_params=pltpu.CompilerParams(use_tc_tiling_on_sc=False),
    )
    out = run(idx, pos, table, pe)
    return out.reshape(BATCH, SEQ_LEN, DIM)
